# baseline (speedup 1.0000x reference)
import functools

import jax
import jax.numpy as jnp
from jax import lax
from jax.experimental import pallas as pl
from jax.experimental.pallas import tpu as pltpu
from jax.experimental.pallas import tpu_sc as plsc

_NC = 2
_NS = 16
_NW = _NC * _NS


@functools.lru_cache(maxsize=None)
def _make_kernel(S, T, L, V, D):
    N = S * T
    tok_w = N // _NW
    C = 4
    ROWS = C * L
    CHUNKS = tok_w // C
    IDXN = tok_w * L

    mesh = plsc.VectorSubcoreMesh(core_axis_name="c", subcore_axis_name="s")

    @functools.partial(
        pl.kernel,
        out_type=jax.ShapeDtypeStruct((N, D), jnp.float32),
        mesh=mesh,
        scratch_types=[
            pltpu.VMEM((IDXN,), jnp.int32),
            pltpu.VMEM((ROWS, D), jnp.float32),
            pltpu.VMEM((C, D), jnp.float32),
            pltpu.SemaphoreType.DMA,
        ],
    )
    def k(idx_hbm, w_hbm, out_hbm, idx_v, rows_v, out_v, sem):
        wid = lax.axis_index("s") * _NC + lax.axis_index("c")
        base_tok = wid * tok_w

        pltpu.sync_copy(idx_hbm.at[wid], idx_v)
        lane = lax.iota(jnp.int32, 16)
        offs = jnp.mod(lane, jnp.int32(L)) * jnp.int32(V)

        def fix(i, _):
            p = i * 16
            idx_v[pl.ds(p, 16)] = idx_v[pl.ds(p, 16)] + offs
            return 0

        lax.fori_loop(0, IDXN // 16, fix, 0)

        def chunk_body(kk, _):
            pltpu.async_copy(
                w_hbm.at[idx_v.at[pl.ds(kk * ROWS, ROWS)]], rows_v, sem
            ).wait()
            for t in range(C):
                def g_body(g, _):
                    p = g * 16
                    acc = rows_v[t * L, pl.ds(p, 16)]
                    for l in range(1, L):
                        acc = acc + rows_v[t * L + l, pl.ds(p, 16)]
                    out_v[t, pl.ds(p, 16)] = acc
                    return 0

                lax.fori_loop(0, D // 16, g_body, 0)
            pltpu.sync_copy(out_v, out_hbm.at[pl.ds(base_tok + kk * C, C)])
            return 0

        lax.fori_loop(0, CHUNKS, chunk_body, 0)

    return k


def kernel(x_list, weight):
    if x_list.shape[0] == 0:
        return ()
    S, T, L = x_list.shape
    Lw, V, D = weight.shape
    N = S * T
    idx = x_list.reshape(_NW, (N * L) // _NW)
    table = weight.reshape(Lw * V, D)
    out = _make_kernel(S, T, L, V, D)(idx, table)
    return tuple(out.reshape(S, T, D)[i] for i in range(S))


# speedup vs baseline: 1.4088x; 1.4088x over previous
import functools

import jax
import jax.numpy as jnp
from jax import lax
from jax.experimental import pallas as pl
from jax.experimental.pallas import tpu as pltpu
from jax.experimental.pallas import tpu_sc as plsc

_NC = 2
_NS = 16
_NW = _NC * _NS


@functools.lru_cache(maxsize=None)
def _make_kernel(S, T, L, V, D):
    N = S * T
    tok_w = N // _NW
    C = 4
    ROWS = C * L
    CHUNKS = tok_w // C
    IDXN = tok_w * L

    mesh = plsc.VectorSubcoreMesh(core_axis_name="c", subcore_axis_name="s")

    @functools.partial(
        pl.kernel,
        out_type=jax.ShapeDtypeStruct((N, D), jnp.float32),
        mesh=mesh,
        scratch_types=[
            pltpu.VMEM((IDXN,), jnp.int32),
            pltpu.VMEM((ROWS, D), jnp.float32),
            pltpu.VMEM((ROWS, D), jnp.float32),
            pltpu.VMEM((C, D), jnp.float32),
            pltpu.VMEM((C, D), jnp.float32),
            pltpu.SemaphoreType.DMA,
            pltpu.SemaphoreType.DMA,
            pltpu.SemaphoreType.DMA,
            pltpu.SemaphoreType.DMA,
        ],
    )
    def k(idx_hbm, w_hbm, out_hbm, idx_v, rows0, rows1, out0, out1,
          sg0, sg1, so0, so1):
        wid = lax.axis_index("s") * _NC + lax.axis_index("c")
        base_tok = wid * tok_w
        rows_b = (rows0, rows1)
        out_b = (out0, out1)
        sg_b = (sg0, sg1)
        so_b = (so0, so1)

        pltpu.sync_copy(idx_hbm.at[wid], idx_v)
        lane = lax.iota(jnp.int32, 16)
        offs = jnp.mod(lane, jnp.int32(L)) * jnp.int32(V)

        def fix(i, _):
            p = i * 16
            idx_v[pl.ds(p, 16)] = idx_v[pl.ds(p, 16)] + offs
            return 0

        lax.fori_loop(0, IDXN // 16, fix, 0)

        def gather_start(kk, b):
            pltpu.async_copy(
                w_hbm.at[idx_v.at[pl.ds(kk * ROWS, ROWS)]], rows_b[b], sg_b[b]
            )

        def gather_wait(kk, b):
            pltpu.make_async_copy(
                w_hbm.at[idx_v.at[pl.ds(kk * ROWS, ROWS)]], rows_b[b], sg_b[b]
            ).wait()

        def compute(kk, b):
            rows_v = rows_b[b]
            out_v = out_b[b]
            for t in range(C):
                def g_body(g, _):
                    base = g * 64
                    for u in range(4):
                        p = base + u * 16
                        acc = rows_v[t * L, pl.ds(p, 16)]
                        for l in range(1, L):
                            acc = acc + rows_v[t * L + l, pl.ds(p, 16)]
                        out_v[t, pl.ds(p, 16)] = acc
                    return 0

                lax.fori_loop(0, D // 64, g_body, 0)

        def out_start(kk, b):
            pltpu.async_copy(
                out_b[b], out_hbm.at[pl.ds(base_tok + kk * C, C)], so_b[b]
            )

        def out_wait(kk, b):
            pltpu.make_async_copy(
                out_b[b], out_hbm.at[pl.ds(base_tok + kk * C, C)], so_b[b]
            ).wait()

        gather_start(0, 0)
        gather_start(1, 1)

        def step(kk, b):
            gather_wait(kk, b)

            @pl.when(kk >= 2)
            def _():
                out_wait(kk - 2, b)

            compute(kk, b)
            gather_start(kk + 2, b)
            out_start(kk, b)

        def outer(i, _):
            k0 = i * 2
            step(k0, 0)
            step(k0 + 1, 1)
            return 0

        lax.fori_loop(0, (CHUNKS - 2) // 2, outer, 0)

        for b, kk in ((0, CHUNKS - 2), (1, CHUNKS - 1)):
            gather_wait(kk, b)
            out_wait(kk - 2, b)
            compute(kk, b)
            out_start(kk, b)
        for b, kk in ((0, CHUNKS - 2), (1, CHUNKS - 1)):
            out_wait(kk, b)

    return k


def kernel(x_list, weight):
    if x_list.shape[0] == 0:
        return ()
    S, T, L = x_list.shape
    Lw, V, D = weight.shape
    N = S * T
    idx = x_list.reshape(_NW, (N * L) // _NW)
    table = weight.reshape(Lw * V, D)
    out = _make_kernel(S, T, L, V, D)(idx, table)
    return tuple(out.reshape(S, T, D)[i] for i in range(S))
